# baseline (speedup 1.0000x reference)
import functools

import jax
import jax.numpy as jnp
from jax import lax
from jax.experimental import pallas as pl
from jax.experimental.pallas import tpu as pltpu
from jax.experimental.pallas import tpu_sc as plsc

_info = plsc.get_sparse_core_info()
_NC, _NS = _info.num_cores, _info.num_subcores
_NW = _NC * _NS

_V = 1000000
_D = 64
_N = 51200
_BPW = _N // _NW
_C = 320
_NCH = _BPW // _C


@jax.jit
def _sc_embed(table2, gidx):
    mesh = plsc.VectorSubcoreMesh(core_axis_name="c", subcore_axis_name="s")

    @functools.partial(
        pl.kernel,
        mesh=mesh,
        out_type=jax.ShapeDtypeStruct((_N, 128), jnp.float32),
        scratch_types=[
            pltpu.VMEM((_C,), jnp.int32),
            pltpu.VMEM((_C, 128), jnp.float32),
            pltpu.SemaphoreType.DMA,
        ],
    )
    def k(tbl_hbm, idx_hbm, out_hbm, idx_v, rows_v, sem):
        wid = lax.axis_index("s") * _NC + lax.axis_index("c")
        base = wid * _BPW

        def body(g, _):
            off = base + g * _C
            pltpu.sync_copy(idx_hbm.at[pl.ds(off, _C)], idx_v)
            pltpu.async_copy(tbl_hbm.at[idx_v], rows_v, sem).wait()
            pltpu.sync_copy(rows_v, out_hbm.at[pl.ds(off, _C)])
            return 0

        lax.fori_loop(0, _NCH, body, 0)

    return k(table2, gidx)


def kernel(encoder_out, encoded_captions, caption_lengths, table):
    b, l = encoded_captions.shape
    flat_idx = encoded_captions.reshape(-1)
    table2 = table.reshape(_V // 2, 2 * _D)
    wide = _sc_embed(table2, flat_idx >> 1)
    out = jnp.where((flat_idx & 1)[:, None] == 1, wide[:, _D:], wide[:, :_D])
    return out.reshape(b, l, _D)
